# Initial kernel scaffold; baseline (speedup 1.0000x reference)
import jax
import jax.numpy as jnp
from jax.experimental import pallas as pl


def kernel(x, edge_index, edge_type, edge_weight, W):
    raise NotImplementedError("write your pallas kernel here")


# speedup vs baseline: 10.2631x; 10.2631x over previous
import functools

import jax
import jax.numpy as jnp
from jax import lax
from jax.experimental import pallas as pl
from jax.experimental.pallas import tpu as pltpu
from jax.experimental.pallas import tpu_sc as plsc

N = 10000
E = 320000
D = 128
R = 8

K = 128
NCHUNK = E // K
NC = 2
NS = 16
NW = NC * NS
CPW = -(-NCHUNK // NW)
NPAD = 10240
ROWS_PER_TILE = NPAD // NS


def _ymm_body(x_ref, w_ref, y_ref):
    y_ref[0] = jnp.dot(x_ref[...], w_ref[0], preferred_element_type=jnp.float32)


def _relation_matmul(x, W):
    BN = 2000
    return pl.pallas_call(
        _ymm_body,
        grid=(R, N // BN),
        in_specs=[
            pl.BlockSpec((BN, D), lambda r, b: (b, 0)),
            pl.BlockSpec((1, D, D), lambda r, b: (r, 0, 0)),
        ],
        out_specs=pl.BlockSpec((1, BN, D), lambda r, b: (r, b, 0)),
        out_shape=jax.ShapeDtypeStruct((R, N, D), jnp.float32),
    )(x, W)


def _make_sc_segment_sum():
    mesh = plsc.VectorSubcoreMesh(core_axis_name="c", subcore_axis_name="s")

    @functools.partial(
        pl.kernel,
        out_type=jax.ShapeDtypeStruct((NC, NPAD, D), jnp.float32),
        mesh=mesh,
        scratch_types=[
            pltpu.VMEM((2, K), jnp.int32),
            pltpu.VMEM((K,), jnp.float32),
            pltpu.VMEM((K, D), jnp.float32),
            pltpu.VMEM_SHARED((NPAD, D), jnp.float32),
            pltpu.SemaphoreType.DMA,
        ],
    )
    def sc_segment_sum(y_hbm, meta_hbm, ew_hbm, out_hbm,
                       meta_v, ew_v, rows_v, acc, sem):
        cid = lax.axis_index("c")
        sid = lax.axis_index("s")
        wid = sid * NC + cid

        def zrow(j, _):
            for i in range(D // 16):
                rows_v[j, pl.ds(i * 16, 16)] = jnp.zeros((16,), jnp.float32)
            return _
        lax.fori_loop(0, K, zrow, None)
        base = sid * ROWS_PER_TILE
        for t in range(ROWS_PER_TILE // K):
            pltpu.sync_copy(rows_v, acc.at[pl.ds(base + t * K, K)])
        plsc.subcore_barrier()

        def chunk_body(j, _):
            c = wid + j * NW

            @pl.when(c < NCHUNK)
            def _():
                pltpu.sync_copy(meta_hbm.at[c], meta_v)
                pltpu.sync_copy(ew_hbm.at[c], ew_v)
                pltpu.async_copy(y_hbm.at[meta_v.at[0]], rows_v, sem).wait()

                def row_body(jb, carry):
                    ew16 = ew_v[pl.ds(jb * 16, 16)]
                    for l in range(16):
                        w = ew16[l]
                        jj = jb * 16 + l
                        for i in range(D // 16):
                            rows_v[jj, pl.ds(i * 16, 16)] = (
                                rows_v[jj, pl.ds(i * 16, 16)] * w)
                    return carry
                lax.fori_loop(0, K // 16, row_body, None)

                pltpu.sync_copy(rows_v, acc.at[meta_v.at[1]], add=True)
            return _
        lax.fori_loop(0, CPW, chunk_body, None)
        plsc.subcore_barrier()

        pltpu.sync_copy(acc.at[pl.ds(base, ROWS_PER_TILE)],
                        out_hbm.at[cid, pl.ds(base, ROWS_PER_TILE)])

    return sc_segment_sum


_sc_segment_sum = _make_sc_segment_sum()


def _epilogue_body(p_ref, x_ref, o_ref):
    m = p_ref[0] + p_ref[1]
    n1 = jnp.sqrt(jnp.sum(m * m, axis=1, keepdims=True))
    m = m / jnp.maximum(n1, 1e-6)
    h = m + x_ref[...]
    n2 = jnp.sqrt(jnp.sum(h * h, axis=1, keepdims=True))
    o_ref[...] = h / n2


def _epilogue(partials, x):
    BN = 2000
    return pl.pallas_call(
        _epilogue_body,
        grid=(N // BN,),
        in_specs=[
            pl.BlockSpec((NC, BN, D), lambda b: (0, b, 0)),
            pl.BlockSpec((BN, D), lambda b: (b, 0)),
        ],
        out_specs=pl.BlockSpec((BN, D), lambda b: (b, 0)),
        out_shape=jax.ShapeDtypeStruct((N, D), jnp.float32),
    )(partials, x)


def kernel(x, edge_index, edge_type, edge_weight, W):
    src = edge_index[0]
    dst = edge_index[1]
    gidx = edge_type * N + src
    meta = jnp.stack([gidx, dst])
    meta = meta.reshape(2, NCHUNK, K).transpose(1, 0, 2)
    ew = edge_weight.reshape(NCHUNK, K)

    y = _relation_matmul(x, W).reshape(R * N, D)
    partials = _sc_segment_sum(y, meta, ew)
    return _epilogue(partials, x)
